# baseline (speedup 1.0000x reference)
import functools

import jax
import jax.numpy as jnp
from jax import lax
from jax.experimental import pallas as pl
from jax.experimental.pallas import tpu as pltpu
from jax.experimental.pallas import tpu_sc as plsc

NC = 2
NS = 16
L = 16


@functools.lru_cache(maxsize=None)
def _build(F, V, D, B):
    ROWS = F * D
    NW = NC * NS
    PER_W = ROWS // NW

    mesh = plsc.VectorSubcoreMesh(core_axis_name="c", subcore_axis_name="s")

    @functools.partial(
        pl.kernel,
        out_type=jax.ShapeDtypeStruct((ROWS, B), jnp.float32),
        mesh=mesh,
        scratch_types=[
            pltpu.VMEM_SHARED((NS, V), jnp.float32),
            pltpu.VMEM((B,), jnp.float32),
        ],
        compiler_params=pltpu.CompilerParams(needs_layout_passes=False),
    )
    def gather_kernel(idx_hbm, tab_hbm, out_hbm, shared, out_v):
        wid = lax.axis_index("s") * NC + lax.axis_index("c")
        sid = lax.axis_index("s")
        c0 = wid * PER_W

        def task(t, carry):
            c = c0 + t
            pltpu.sync_copy(tab_hbm.at[c, :], shared.at[sid])
            return carry

        lax.fori_loop(0, PER_W, task, 0)
        pltpu.sync_copy(out_v, out_hbm.at[c0, :])

    return gather_kernel


def kernel(indices, tables):
    B, F = indices.shape
    F2, V, D = tables.shape
    idx_t = indices.T
    tab2 = jnp.transpose(tables, (0, 2, 1)).reshape(F * D, V)
    out_t = _build(F, V, D, B)(idx_t, tab2)
    return out_t.T


# speedup vs baseline: 1.0971x; 1.0971x over previous
import functools

import jax
import jax.numpy as jnp
from jax import lax
from jax.experimental import pallas as pl
from jax.experimental.pallas import tpu as pltpu
from jax.experimental.pallas import tpu_sc as plsc

NC = 2
NS = 16
L = 16
OC = 4096


@functools.lru_cache(maxsize=None)
def _build(F, V, D, B):
    ROWS = F * D
    NW = NC * NS
    assert ROWS % NW == 0
    PER_W = ROWS // NW
    NK = B // OC
    assert NK >= 2 and OC % L == 0

    mesh = plsc.VectorSubcoreMesh(core_axis_name="c", subcore_axis_name="s")

    @functools.partial(
        pl.kernel,
        out_type=jax.ShapeDtypeStruct((ROWS, B), jnp.float32),
        mesh=mesh,
        scratch_types=[
            pltpu.VMEM((V,), jnp.float32),
            pltpu.VMEM((B,), jnp.int32),
            pltpu.VMEM((OC,), jnp.float32),
            pltpu.VMEM((OC,), jnp.float32),
            pltpu.SemaphoreType.DMA,
        ],
        compiler_params=pltpu.CompilerParams(needs_layout_passes=False),
    )
    def gather_kernel(idx_hbm, tab_hbm, out_hbm, row_v, idx_v, out_a, out_b,
                      sem_o):
        wid = lax.axis_index("s") * NC + lax.axis_index("c")
        c0 = wid * PER_W

        def wait_out(c):
            pltpu.make_async_copy(
                out_a, out_hbm.at[c, pl.ds(0, OC)], sem_o).wait()

        def task(t, carry):
            c = c0 + t
            f = c // D
            d = c % D
            pltpu.sync_copy(tab_hbm.at[c, :], row_v)

            @pl.when(jnp.logical_or(t == 0, d == 0))
            def _():
                pltpu.sync_copy(idx_hbm.at[f, :], idx_v)

            for k in range(NK):
                buf = out_a if k % 2 == 0 else out_b
                if k >= 2:
                    wait_out(c)
                else:
                    @pl.when(t > 0)
                    def _():
                        wait_out(c)

                @plsc.parallel_loop(0, OC, step=L, unroll=8)
                def gbody(i):
                    ids = idx_v[pl.ds(k * OC + i, L)]
                    buf[pl.ds(i, L)] = plsc.load_gather(row_v, [ids])

                pltpu.async_copy(buf, out_hbm.at[c, pl.ds(k * OC, OC)], sem_o)
            return carry

        lax.fori_loop(0, PER_W, task, 0)
        wait_out(c0)
        wait_out(c0)

    return gather_kernel


def kernel(indices, tables):
    B, F = indices.shape
    F2, V, D = tables.shape
    idx_t = indices.T
    tab2 = jnp.transpose(tables, (0, 2, 1)).reshape(F * D, V)
    out_t = _build(F, V, D, B)(idx_t, tab2)
    return out_t.T


# speedup vs baseline: 1.0980x; 1.0008x over previous
import functools

import jax
import jax.numpy as jnp
from jax import lax
from jax.experimental import pallas as pl
from jax.experimental.pallas import tpu as pltpu
from jax.experimental.pallas import tpu_sc as plsc

NC = 2
NS = 16
L = 16
OC = 4096


@functools.lru_cache(maxsize=None)
def _build(F, V, D, B):
    ROWS = F * D
    NW = NC * NS
    assert ROWS % NW == 0
    PER_W = ROWS // NW
    NK = B // OC
    assert NK >= 2 and OC % L == 0

    mesh = plsc.VectorSubcoreMesh(core_axis_name="c", subcore_axis_name="s")

    @functools.partial(
        pl.kernel,
        out_type=jax.ShapeDtypeStruct((ROWS, B), jnp.float32),
        mesh=mesh,
        scratch_types=[
            pltpu.VMEM((V,), jnp.float32),
            pltpu.VMEM((B,), jnp.int32),
            pltpu.VMEM((OC,), jnp.float32),
            pltpu.VMEM((OC,), jnp.float32),
            pltpu.SemaphoreType.DMA,
            pltpu.SemaphoreType.DMA,
        ],
        compiler_params=pltpu.CompilerParams(needs_layout_passes=False),
    )
    def gather_kernel(idx_hbm, tab_hbm, out_hbm, row_v, idx_v, out_a, out_b,
                      sem_a, sem_b):
        wid = lax.axis_index("s") * NC + lax.axis_index("c")
        c0 = wid * PER_W

        def wait_out(c, sem):
            pltpu.make_async_copy(
                out_a, out_hbm.at[c, pl.ds(0, OC)], sem).wait()

        def task(t, carry):
            c = c0 + t
            f = c // D
            d = c % D
            pltpu.sync_copy(tab_hbm.at[c, :], row_v)

            @pl.when(jnp.logical_or(t == 0, d == 0))
            def _():
                pltpu.sync_copy(idx_hbm.at[f, :], idx_v)

            for k in range(NK):
                buf, sem = (out_a, sem_a) if k % 2 == 0 else (out_b, sem_b)
                if k >= 2:
                    wait_out(c, sem)
                else:
                    @pl.when(t > 0)
                    def _():
                        wait_out(c, sem)

                @plsc.parallel_loop(0, OC, step=L, unroll=8)
                def gbody(i):
                    ids = idx_v[pl.ds(k * OC + i, L)]
                    buf[pl.ds(i, L)] = plsc.load_gather(row_v, [ids])

                pltpu.async_copy(buf, out_hbm.at[c, pl.ds(k * OC, OC)], sem)
            return carry

        lax.fori_loop(0, PER_W, task, 0)
        wait_out(c0, sem_a)
        wait_out(c0, sem_b)

    return gather_kernel


def kernel(indices, tables):
    B, F = indices.shape
    F2, V, D = tables.shape
    idx_t = indices.T
    tab2 = jnp.transpose(tables, (0, 2, 1)).reshape(F * D, V)
    out_t = _build(F, V, D, B)(idx_t, tab2)
    return out_t.T
